# baseline (speedup 1.0000x reference)
import functools

import jax
import jax.numpy as jnp
from jax.experimental import pallas as pl

EPS = 1e-16


def _cos_body(mem_ref, k_ref, cos_ref):
    mem = mem_ref[...] + EPS
    kk = k_ref[...] + EPS
    num = jnp.sum(mem * kk[:, None, :], axis=-1)
    ssq = jnp.sum(mem * mem, axis=-1)
    normk = jnp.sqrt(jnp.sum(kk * kk, axis=-1))
    denom = jnp.sqrt(ssq) * normk[:, None]
    cos_ref[...] = num / jnp.maximum(denom, 1e-8)


def _w_body(cos_ref, wprev_ref, beta_ref, g_ref, s_ref, gamma_ref, w_ref):
    cos = cos_ref[...]
    beta = beta_ref[...]
    x = beta * cos
    x = x - jnp.max(x, axis=1, keepdims=True)
    ex = jnp.exp(x)
    wc = ex / jnp.sum(ex, axis=1, keepdims=True)
    g = g_ref[...]
    wg = g * wc + (1.0 - g) * wprev_ref[...]
    s = s_ref[...]
    left = jnp.concatenate([wg[:, -1:], wg[:, :-1]], axis=1)
    right = jnp.concatenate([wg[:, 1:], wg[:, :1]], axis=1)
    sh = left * s[:, 0:1] + wg * s[:, 1:2] + right * s[:, 2:3]
    gamma = gamma_ref[...]
    wpow = jnp.where(sh > 0.0,
                     jnp.exp(gamma * jnp.log(jnp.maximum(sh, 1e-38))),
                     0.0)
    w_ref[...] = wpow / (jnp.sum(wpow, axis=1, keepdims=True) + EPS)


def _read_body(w_ref, mem_ref, out_ref):
    @pl.when(pl.program_id(0) == 0)
    def _():
        out_ref[...] = jnp.zeros_like(out_ref)

    w = w_ref[...]
    mem = mem_ref[...]
    out_ref[...] += jnp.sum(mem * w[:, :, None], axis=1)


@jax.jit
def kernel(memory, k, beta, g, s, gamma, w_prev):
    B, N, M = memory.shape
    BN = min(512, N)
    nb = N // BN

    cos = pl.pallas_call(
        _cos_body,
        grid=(nb,),
        in_specs=[
            pl.BlockSpec((B, BN, M), lambda i: (0, i, 0)),
            pl.BlockSpec((B, M), lambda i: (0, 0)),
        ],
        out_specs=pl.BlockSpec((B, BN), lambda i: (0, i)),
        out_shape=jax.ShapeDtypeStruct((B, N), jnp.float32),
    )(memory, k)

    w = pl.pallas_call(
        _w_body,
        in_specs=[pl.BlockSpec(x.shape, lambda: (0,) * x.ndim)
                  for x in (cos, w_prev, beta, g, s, gamma)],
        out_specs=pl.BlockSpec((B, N), lambda: (0, 0)),
        out_shape=jax.ShapeDtypeStruct((B, N), jnp.float32),
    )(cos, w_prev, beta, g, s, gamma)

    read = pl.pallas_call(
        _read_body,
        grid=(nb,),
        in_specs=[
            pl.BlockSpec((B, BN), lambda i: (0, i)),
            pl.BlockSpec((B, BN, M), lambda i: (0, i, 0)),
        ],
        out_specs=pl.BlockSpec((B, M), lambda i: (0, 0)),
        out_shape=jax.ShapeDtypeStruct((B, M), jnp.float32),
    )(w, memory)

    return read


# speedup vs baseline: 1.3748x; 1.3748x over previous
import functools

import jax
import jax.numpy as jnp
from jax.experimental import pallas as pl

EPS = 1e-16


def _cos_body(mem_ref, k_ref, cos_ref):
    mem = mem_ref[...] + EPS
    kk = k_ref[...] + EPS
    mem_t = jnp.swapaxes(mem, 1, 2)
    kk_r = kk[:, :, None]
    num = jnp.sum(mem_t * kk_r, axis=1)
    ssq = jnp.sum(mem_t * mem_t, axis=1)
    normk = jnp.sqrt(jnp.sum(kk * kk, axis=-1))
    denom = jnp.sqrt(ssq) * normk[:, None]
    cos_ref[...] = num / jnp.maximum(denom, 1e-8)


def _w_body(cos_ref, wprev_ref, beta_ref, g_ref, s_ref, gamma_ref, w_ref):
    cos = cos_ref[...]
    beta = beta_ref[...]
    x = beta * cos
    x = x - jnp.max(x, axis=1, keepdims=True)
    ex = jnp.exp(x)
    wc = ex / jnp.sum(ex, axis=1, keepdims=True)
    g = g_ref[...]
    wg = g * wc + (1.0 - g) * wprev_ref[...]
    s = s_ref[...]
    left = jnp.concatenate([wg[:, -1:], wg[:, :-1]], axis=1)
    right = jnp.concatenate([wg[:, 1:], wg[:, :1]], axis=1)
    sh = left * s[:, 0:1] + wg * s[:, 1:2] + right * s[:, 2:3]
    gamma = gamma_ref[...]
    wpow = jnp.where(sh > 0.0,
                     jnp.exp(gamma * jnp.log(jnp.maximum(sh, 1e-38))),
                     0.0)
    w_ref[...] = wpow / (jnp.sum(wpow, axis=1, keepdims=True) + EPS)


def _read_body(w_ref, mem_ref, out_ref):
    @pl.when(pl.program_id(0) == 0)
    def _():
        out_ref[...] = jnp.zeros_like(out_ref)

    w = w_ref[...]
    mem = mem_ref[...]
    out_ref[...] += jnp.sum(mem * w[:, :, None], axis=1)


@jax.jit
def kernel(memory, k, beta, g, s, gamma, w_prev):
    B, N, M = memory.shape
    BN = min(512, N)
    nb = N // BN

    cos = pl.pallas_call(
        _cos_body,
        grid=(nb,),
        in_specs=[
            pl.BlockSpec((B, BN, M), lambda i: (0, i, 0)),
            pl.BlockSpec((B, M), lambda i: (0, 0)),
        ],
        out_specs=pl.BlockSpec((B, BN), lambda i: (0, i)),
        out_shape=jax.ShapeDtypeStruct((B, N), jnp.float32),
    )(memory, k)

    w = pl.pallas_call(
        _w_body,
        in_specs=[pl.BlockSpec(x.shape, lambda: (0,) * x.ndim)
                  for x in (cos, w_prev, beta, g, s, gamma)],
        out_specs=pl.BlockSpec((B, N), lambda: (0, 0)),
        out_shape=jax.ShapeDtypeStruct((B, N), jnp.float32),
    )(cos, w_prev, beta, g, s, gamma)

    read = pl.pallas_call(
        _read_body,
        grid=(nb,),
        in_specs=[
            pl.BlockSpec((B, BN), lambda i: (0, i)),
            pl.BlockSpec((B, BN, M), lambda i: (0, i, 0)),
        ],
        out_specs=pl.BlockSpec((B, M), lambda i: (0, 0)),
        out_shape=jax.ShapeDtypeStruct((B, M), jnp.float32),
    )(w, memory)

    return read


# speedup vs baseline: 1.4230x; 1.0350x over previous
import functools

import jax
import jax.numpy as jnp
from jax.experimental import pallas as pl

EPS = 1e-16


def _cos_body(mem_ref, k_ref, cos_ref):
    mem = mem_ref[...]
    kk = k_ref[...] + EPS
    mem_t = jnp.swapaxes(mem, 1, 2)
    kk_r = kk[:, :, None]
    num = jnp.sum(mem_t * kk_r, axis=1)
    ssq = jnp.sum(mem_t * mem_t, axis=1)
    normk = jnp.sqrt(jnp.sum(kk * kk, axis=-1))
    denom = jnp.sqrt(ssq) * normk[:, None]
    cos_ref[...] = num / jnp.maximum(denom, 1e-8)


def _w_body(cos_ref, wprev_ref, beta_ref, g_ref, s_ref, gamma_ref, w_ref):
    cos = cos_ref[...]
    beta = beta_ref[...]
    x = beta * cos
    x = x - jnp.max(x, axis=1, keepdims=True)
    ex = jnp.exp(x)
    wc = ex / jnp.sum(ex, axis=1, keepdims=True)
    g = g_ref[...]
    wg = g * wc + (1.0 - g) * wprev_ref[...]
    s = s_ref[...]
    left = jnp.concatenate([wg[:, -1:], wg[:, :-1]], axis=1)
    right = jnp.concatenate([wg[:, 1:], wg[:, :1]], axis=1)
    sh = left * s[:, 0:1] + wg * s[:, 1:2] + right * s[:, 2:3]
    gamma = gamma_ref[...]
    wpow = jnp.where(sh > 0.0,
                     jnp.exp(gamma * jnp.log(jnp.maximum(sh, 1e-38))),
                     0.0)
    w_ref[...] = wpow / (jnp.sum(wpow, axis=1, keepdims=True) + EPS)


def _read_body(w_ref, mem_ref, out_ref):
    @pl.when(pl.program_id(0) == 0)
    def _():
        out_ref[...] = jnp.zeros_like(out_ref)

    w = w_ref[...]
    mem = mem_ref[...]
    acc = jnp.concatenate(
        [jax.lax.dot(w[b:b + 1, :], mem[b]) for b in range(mem.shape[0])],
        axis=0)
    out_ref[...] += acc


@jax.jit
def kernel(memory, k, beta, g, s, gamma, w_prev):
    B, N, M = memory.shape
    BN = min(512, N)
    nb = N // BN

    cos = pl.pallas_call(
        _cos_body,
        grid=(nb,),
        in_specs=[
            pl.BlockSpec((B, BN, M), lambda i: (0, i, 0)),
            pl.BlockSpec((B, M), lambda i: (0, 0)),
        ],
        out_specs=pl.BlockSpec((B, BN), lambda i: (0, i)),
        out_shape=jax.ShapeDtypeStruct((B, N), jnp.float32),
    )(memory, k)

    w = pl.pallas_call(
        _w_body,
        in_specs=[pl.BlockSpec(x.shape, lambda: (0,) * x.ndim)
                  for x in (cos, w_prev, beta, g, s, gamma)],
        out_specs=pl.BlockSpec((B, N), lambda: (0, 0)),
        out_shape=jax.ShapeDtypeStruct((B, N), jnp.float32),
    )(cos, w_prev, beta, g, s, gamma)

    read = pl.pallas_call(
        _read_body,
        grid=(nb,),
        in_specs=[
            pl.BlockSpec((B, BN), lambda i: (0, i)),
            pl.BlockSpec((B, BN, M), lambda i: (0, i, 0)),
        ],
        out_specs=pl.BlockSpec((B, M), lambda i: (0, 0)),
        out_shape=jax.ShapeDtypeStruct((B, M), jnp.float32),
    )(w, memory)

    return read


# speedup vs baseline: 4.9424x; 3.4734x over previous
import jax
import jax.numpy as jnp
from jax.experimental import pallas as pl

EPS = 1e-16


def _cos_body(memt_ref, k_ref, cos_ref):
    memt = memt_ref[...]
    kk = k_ref[...] + EPS
    num = jnp.sum(memt * kk[:, :, None], axis=1)
    ssq = jnp.sum(memt * memt, axis=1)
    normk = jnp.sqrt(jnp.sum(kk * kk, axis=-1))
    denom = jnp.sqrt(ssq) * normk[:, None]
    cos_ref[...] = num / jnp.maximum(denom, 1e-8)


def _w_body(cos_ref, wprev_ref, beta_ref, g_ref, s_ref, gamma_ref, w_ref):
    cos = cos_ref[...]
    beta = beta_ref[...]
    x = beta * cos
    x = x - jnp.max(x, axis=1, keepdims=True)
    ex = jnp.exp(x)
    wc = ex / jnp.sum(ex, axis=1, keepdims=True)
    g = g_ref[...]
    wg = g * wc + (1.0 - g) * wprev_ref[...]
    s = s_ref[...]
    left = jnp.concatenate([wg[:, -1:], wg[:, :-1]], axis=1)
    right = jnp.concatenate([wg[:, 1:], wg[:, :1]], axis=1)
    sh = left * s[:, 0:1] + wg * s[:, 1:2] + right * s[:, 2:3]
    gamma = gamma_ref[...]
    wpow = jnp.where(sh > 0.0,
                     jnp.exp(gamma * jnp.log(jnp.maximum(sh, 1e-38))),
                     0.0)
    w_ref[...] = wpow / (jnp.sum(wpow, axis=1, keepdims=True) + EPS)


def _read_body(w_ref, memt_ref, out_ref):
    @pl.when(pl.program_id(0) == 0)
    def _():
        out_ref[...] = jnp.zeros_like(out_ref)

    w = w_ref[...]
    memt = memt_ref[...]
    out_ref[...] += jnp.sum(memt * w[:, None, :], axis=2)


@jax.jit
def kernel(memory, k, beta, g, s, gamma, w_prev):
    B, N, M = memory.shape
    BN = min(512, N)
    nb = N // BN
    memt = jnp.swapaxes(memory, 1, 2)

    cos = pl.pallas_call(
        _cos_body,
        grid=(nb,),
        in_specs=[
            pl.BlockSpec((B, M, BN), lambda i: (0, 0, i)),
            pl.BlockSpec((B, M), lambda i: (0, 0)),
        ],
        out_specs=pl.BlockSpec((B, BN), lambda i: (0, i)),
        out_shape=jax.ShapeDtypeStruct((B, N), jnp.float32),
    )(memt, k)

    w = pl.pallas_call(
        _w_body,
        in_specs=[pl.BlockSpec(x.shape, lambda: (0,) * x.ndim)
                  for x in (cos, w_prev, beta, g, s, gamma)],
        out_specs=pl.BlockSpec((B, N), lambda: (0, 0)),
        out_shape=jax.ShapeDtypeStruct((B, N), jnp.float32),
    )(cos, w_prev, beta, g, s, gamma)

    read = pl.pallas_call(
        _read_body,
        grid=(nb,),
        in_specs=[
            pl.BlockSpec((B, BN), lambda i: (0, i)),
            pl.BlockSpec((B, M, BN), lambda i: (0, 0, i)),
        ],
        out_specs=pl.BlockSpec((B, M), lambda i: (0, 0)),
        out_shape=jax.ShapeDtypeStruct((B, M), jnp.float32),
    )(w, memt)

    return read


# speedup vs baseline: 5.3629x; 1.0851x over previous
import jax
import jax.numpy as jnp
from jax.experimental import pallas as pl

EPS = 1e-16


def _cos_body(memt_ref, k_ref, cos_ref):
    memt = memt_ref[...]
    kk = k_ref[...] + EPS
    num = jnp.sum(memt * kk[:, :, None], axis=1)
    ssq = jnp.sum(memt * memt, axis=1)
    normk = jnp.sqrt(jnp.sum(kk * kk, axis=-1))
    denom = jnp.sqrt(ssq) * normk[:, None]
    cos_ref[...] = num / jnp.maximum(denom, 1e-8)


def _w_body(cos_ref, wprev_ref, beta_ref, g_ref, s_ref, gamma_ref, w_ref):
    cos = cos_ref[...]
    beta = beta_ref[...]
    x = beta * cos
    x = x - jnp.max(x, axis=1, keepdims=True)
    ex = jnp.exp(x)
    wc = ex / jnp.sum(ex, axis=1, keepdims=True)
    g = g_ref[...]
    wg = g * wc + (1.0 - g) * wprev_ref[...]
    s = s_ref[...]
    left = jnp.concatenate([wg[:, -1:], wg[:, :-1]], axis=1)
    right = jnp.concatenate([wg[:, 1:], wg[:, :1]], axis=1)
    sh = left * s[:, 0:1] + wg * s[:, 1:2] + right * s[:, 2:3]
    gamma = gamma_ref[...]
    wpow = jnp.where(sh > 0.0,
                     jnp.exp(gamma * jnp.log(jnp.maximum(sh, 1e-38))),
                     0.0)
    w_ref[...] = wpow / (jnp.sum(wpow, axis=1, keepdims=True) + EPS)


def _read_body(w_ref, memt_ref, out_ref):
    @pl.when(pl.program_id(0) == 0)
    def _():
        out_ref[...] = jnp.zeros_like(out_ref)

    w = w_ref[...]
    memt = memt_ref[...]
    out_ref[...] += jnp.sum(memt * w[:, None, :], axis=2)


@jax.jit
def kernel(memory, k, beta, g, s, gamma, w_prev):
    B, N, M = memory.shape
    BN = min(1024, N)
    nb = N // BN
    memt = jnp.swapaxes(memory, 1, 2)

    cos = pl.pallas_call(
        _cos_body,
        grid=(nb,),
        in_specs=[
            pl.BlockSpec((B, M, BN), lambda i: (0, 0, i)),
            pl.BlockSpec((B, M), lambda i: (0, 0)),
        ],
        out_specs=pl.BlockSpec((B, BN), lambda i: (0, i)),
        out_shape=jax.ShapeDtypeStruct((B, N), jnp.float32),
    )(memt, k)

    w = pl.pallas_call(
        _w_body,
        in_specs=[pl.BlockSpec(x.shape, lambda: (0,) * x.ndim)
                  for x in (cos, w_prev, beta, g, s, gamma)],
        out_specs=pl.BlockSpec((B, N), lambda: (0, 0)),
        out_shape=jax.ShapeDtypeStruct((B, N), jnp.float32),
    )(cos, w_prev, beta, g, s, gamma)

    read = pl.pallas_call(
        _read_body,
        grid=(nb,),
        in_specs=[
            pl.BlockSpec((B, BN), lambda i: (0, i)),
            pl.BlockSpec((B, M, BN), lambda i: (0, 0, i)),
        ],
        out_specs=pl.BlockSpec((B, M), lambda i: (0, 0)),
        out_shape=jax.ShapeDtypeStruct((B, M), jnp.float32),
    )(w, memt)

    return read


# speedup vs baseline: 5.4282x; 1.0122x over previous
import jax
import jax.numpy as jnp
from jax.experimental import pallas as pl

EPS = 1e-16


def _cos_body(memt_ref, k_ref, cos_ref):
    memt = memt_ref[...]
    kk = k_ref[...] + EPS
    num = jnp.sum(memt * kk[:, :, None], axis=1)
    ssq = jnp.sum(memt * memt, axis=1)
    normk = jnp.sqrt(jnp.sum(kk * kk, axis=-1))
    denom = jnp.sqrt(ssq) * normk[:, None]
    cos_ref[...] = num / jnp.maximum(denom, 1e-8)


def _w_body(cos_ref, wprev_ref, beta_ref, g_ref, s_ref, gamma_ref, w_ref):
    cos = cos_ref[...]
    beta = beta_ref[...]
    x = beta * cos
    x = x - jnp.max(x, axis=1, keepdims=True)
    ex = jnp.exp(x)
    wc = ex / jnp.sum(ex, axis=1, keepdims=True)
    g = g_ref[...]
    wg = g * wc + (1.0 - g) * wprev_ref[...]
    s = s_ref[...]
    left = jnp.concatenate([wg[:, -1:], wg[:, :-1]], axis=1)
    right = jnp.concatenate([wg[:, 1:], wg[:, :1]], axis=1)
    sh = left * s[:, 0:1] + wg * s[:, 1:2] + right * s[:, 2:3]
    gamma = gamma_ref[...]
    wpow = jnp.where(sh > 0.0,
                     jnp.exp(gamma * jnp.log(jnp.maximum(sh, 1e-38))),
                     0.0)
    w_ref[...] = wpow / (jnp.sum(wpow, axis=1, keepdims=True) + EPS)


def _read_body(w_ref, memt_ref, out_ref):
    @pl.when(pl.program_id(0) == 0)
    def _():
        out_ref[...] = jnp.zeros_like(out_ref)

    w = w_ref[...]
    memt = memt_ref[...]
    out_ref[...] += jnp.sum(memt * w[:, None, :], axis=2)


@jax.jit
def kernel(memory, k, beta, g, s, gamma, w_prev):
    B, N, M = memory.shape
    BN = min(2048, N)
    nb = N // BN
    memt = jnp.swapaxes(memory, 1, 2)

    cos = pl.pallas_call(
        _cos_body,
        grid=(nb,),
        in_specs=[
            pl.BlockSpec((B, M, BN), lambda i: (0, 0, i)),
            pl.BlockSpec((B, M), lambda i: (0, 0)),
        ],
        out_specs=pl.BlockSpec((B, BN), lambda i: (0, i)),
        out_shape=jax.ShapeDtypeStruct((B, N), jnp.float32),
    )(memt, k)

    w = pl.pallas_call(
        _w_body,
        in_specs=[pl.BlockSpec(x.shape, lambda: (0,) * x.ndim)
                  for x in (cos, w_prev, beta, g, s, gamma)],
        out_specs=pl.BlockSpec((B, N), lambda: (0, 0)),
        out_shape=jax.ShapeDtypeStruct((B, N), jnp.float32),
    )(cos, w_prev, beta, g, s, gamma)

    read = pl.pallas_call(
        _read_body,
        grid=(nb,),
        in_specs=[
            pl.BlockSpec((B, BN), lambda i: (0, i)),
            pl.BlockSpec((B, M, BN), lambda i: (0, 0, i)),
        ],
        out_specs=pl.BlockSpec((B, M), lambda i: (0, 0)),
        out_shape=jax.ShapeDtypeStruct((B, M), jnp.float32),
    )(w, memt)

    return read
